# baseline (speedup 1.0000x reference)
import functools
import jax
import jax.numpy as jnp
from jax import lax
from jax.experimental import pallas as pl
from jax.experimental.pallas import tpu as pltpu
from jax.experimental.pallas import tpu_sc as plsc

B, I, F = 4096, 16, 512
L = 16
NC, NS = 2, 16
NW = NC * NS
TOT = B * I * F
SLICE = TOT // NW
CH_B = 2
CHUNK = CH_B * I * F
NCHUNK = SLICE // CHUNK
ROWS = CH_B * I
NJ = F // L


def _sc_body(o_hbm, l_hbm, imp_hbm, part_hbm,
             ob0, ob1, lb0, lb1, impv, accv,
             so0, so1, sl0, sl1):
    wid = lax.axis_index("s") * NC + lax.axis_index("c")
    base = wid * SLICE

    pltpu.sync_copy(imp_hbm, impv)
    for k in range(I):
        accv[pl.ds(k * L, L)] = jnp.zeros((L,), jnp.float32)

    obufs, lbufs = (ob0, ob1), (lb0, lb1)
    osems, lsems = (so0, so1), (sl0, sl1)

    for s in range(2):
        off = base + s * CHUNK
        pltpu.async_copy(o_hbm.at[pl.ds(off, CHUNK)], obufs[s], osems[s])
        pltpu.async_copy(l_hbm.at[pl.ds(off, CHUNK)], lbufs[s], lsems[s])

    def pair_body(p, carry):
        for s in range(2):
            c = p * 2 + s
            off = base + c * CHUNK
            ob, lb = obufs[s], lbufs[s]
            pltpu.make_async_copy(o_hbm.at[pl.ds(off, CHUNK)], ob, osems[s]).wait()
            pltpu.make_async_copy(l_hbm.at[pl.ds(off, CHUNK)], lb, lsems[s]).wait()

            def row_body(r, _, ob=ob, lb=lb):
                i = lax.rem(r, I)
                ioff = i * F
                roff = r * F
                vacc = accv[pl.ds(i * L, L)]
                for j in range(NJ):
                    o = ob[pl.ds(roff + j * L, L)]
                    l = lb[pl.ds(roff + j * L, L)]
                    m = impv[pl.ds(ioff + j * L, L)]
                    d = m * (jnp.abs(l) - o)
                    vacc = vacc + d * d
                accv[pl.ds(i * L, L)] = vacc
                return 0

            lax.fori_loop(0, ROWS, row_body, 0)

            @pl.when(c + 2 < NCHUNK)
            def _():
                off2 = base + (c + 2) * CHUNK
                pltpu.async_copy(o_hbm.at[pl.ds(off2, CHUNK)], obufs[s], osems[s])
                pltpu.async_copy(l_hbm.at[pl.ds(off2, CHUNK)], lbufs[s], lsems[s])
        return carry

    lax.fori_loop(0, NCHUNK // 2, pair_body, 0)

    pltpu.sync_copy(accv, part_hbm.at[wid])


@functools.cache
def _build_sc():
    mesh = plsc.VectorSubcoreMesh(core_axis_name="c", subcore_axis_name="s")
    return pl.kernel(
        _sc_body,
        out_type=jax.ShapeDtypeStruct((NW, I * L), jnp.float32),
        mesh=mesh,
        scratch_types=[
            pltpu.VMEM((CHUNK,), jnp.float32),
            pltpu.VMEM((CHUNK,), jnp.float32),
            pltpu.VMEM((CHUNK,), jnp.float32),
            pltpu.VMEM((CHUNK,), jnp.float32),
            pltpu.VMEM((I * F,), jnp.float32),
            pltpu.VMEM((I * L,), jnp.float32),
            pltpu.SemaphoreType.DMA,
            pltpu.SemaphoreType.DMA,
            pltpu.SemaphoreType.DMA,
            pltpu.SemaphoreType.DMA,
        ],
    )


def _fold_kernel(p_ref, o_ref):
    s = jnp.sum(jnp.sum(p_ref[...], axis=0), axis=1)
    o_ref[0, :] = s * (1.0 / (B * F))


def _fold(partials):
    return pl.pallas_call(
        _fold_kernel,
        in_specs=[pl.BlockSpec((NW, I, L), lambda: (0, 0, 0))],
        out_specs=pl.BlockSpec((1, I), lambda: (0, 0)),
        out_shape=jax.ShapeDtypeStruct((1, I), jnp.float32),
    )(partials)[0]


def kernel(out, labels, importance):
    o = out.reshape(TOT)
    l = labels.reshape(TOT)
    imp = importance.reshape(I * F)
    partials = _build_sc()(o, l, imp)
    return _fold(partials.reshape(NW, I, L))


# speedup vs baseline: 1.1806x; 1.1806x over previous
import functools
import jax
import jax.numpy as jnp
from jax import lax
from jax.experimental import pallas as pl
from jax.experimental.pallas import tpu as pltpu
from jax.experimental.pallas import tpu_sc as plsc

B, I, F = 4096, 16, 512
L = 16
NC, NS = 2, 16
NW = NC * NS

SC_B = 768
TC_B = B - SC_B

SC_ELEMS = SC_B * I * F
SLICE = SC_ELEMS // NW
CH_B = 2
CHUNK = CH_B * I * F
NCHUNK = SLICE // CHUNK
ROWS = CH_B * I
NJ = F // L
assert SLICE % CHUNK == 0 and NCHUNK % 2 == 0

BB = 256
TC_GRID = TC_B // BB
assert TC_B % BB == 0 and SC_B % BB == 0


def _sc_body(o_hbm, l_hbm, imp_hbm, part_hbm,
             ob0, ob1, lb0, lb1, impv, accv,
             so0, so1, sl0, sl1):
    wid = lax.axis_index("s") * NC + lax.axis_index("c")
    base = wid * SLICE

    pltpu.sync_copy(imp_hbm, impv)
    for k in range(I):
        accv[pl.ds(k * L, L)] = jnp.zeros((L,), jnp.float32)

    obufs, lbufs = (ob0, ob1), (lb0, lb1)
    osems, lsems = (so0, so1), (sl0, sl1)

    for s in range(2):
        off = base + s * CHUNK
        pltpu.async_copy(o_hbm.at[pl.ds(off, CHUNK)], obufs[s], osems[s])
        pltpu.async_copy(l_hbm.at[pl.ds(off, CHUNK)], lbufs[s], lsems[s])

    def pair_body(p, carry):
        for s in range(2):
            c = p * 2 + s
            off = base + c * CHUNK
            ob, lb = obufs[s], lbufs[s]
            pltpu.make_async_copy(o_hbm.at[pl.ds(off, CHUNK)], ob, osems[s]).wait()
            pltpu.make_async_copy(l_hbm.at[pl.ds(off, CHUNK)], lb, lsems[s]).wait()

            def row_body(r, _, ob=ob, lb=lb):
                i = lax.rem(r, I)
                ioff = i * F
                roff = r * F
                vacc = accv[pl.ds(i * L, L)]
                for j in range(NJ):
                    o = ob[pl.ds(roff + j * L, L)]
                    l = lb[pl.ds(roff + j * L, L)]
                    m = impv[pl.ds(ioff + j * L, L)]
                    d = m * (jnp.abs(l) - o)
                    vacc = vacc + d * d
                accv[pl.ds(i * L, L)] = vacc
                return 0

            lax.fori_loop(0, ROWS, row_body, 0)

            @pl.when(c + 2 < NCHUNK)
            def _():
                off2 = base + (c + 2) * CHUNK
                pltpu.async_copy(o_hbm.at[pl.ds(off2, CHUNK)], obufs[s], osems[s])
                pltpu.async_copy(l_hbm.at[pl.ds(off2, CHUNK)], lbufs[s], lsems[s])
        return carry

    lax.fori_loop(0, NCHUNK // 2, pair_body, 0)

    pltpu.sync_copy(accv, part_hbm.at[wid])


@functools.cache
def _build_sc():
    mesh = plsc.VectorSubcoreMesh(core_axis_name="c", subcore_axis_name="s")
    return pl.kernel(
        _sc_body,
        out_type=jax.ShapeDtypeStruct((NW, I * L), jnp.float32),
        mesh=mesh,
        scratch_types=[
            pltpu.VMEM((CHUNK,), jnp.float32),
            pltpu.VMEM((CHUNK,), jnp.float32),
            pltpu.VMEM((CHUNK,), jnp.float32),
            pltpu.VMEM((CHUNK,), jnp.float32),
            pltpu.VMEM((I * F,), jnp.float32),
            pltpu.VMEM((I * L,), jnp.float32),
            pltpu.SemaphoreType.DMA,
            pltpu.SemaphoreType.DMA,
            pltpu.SemaphoreType.DMA,
            pltpu.SemaphoreType.DMA,
        ],
    )


def _tc_kernel(out_ref, lab_ref, imp_ref, o_ref, acc_ref):
    step = pl.program_id(0)
    d = imp_ref[...] * (jnp.abs(lab_ref[...]) - out_ref[...])
    partial = jnp.sum(d * d, axis=0)

    @pl.when(step == 0)
    def _():
        acc_ref[...] = partial

    @pl.when(step > 0)
    def _():
        acc_ref[...] = acc_ref[...] + partial

    @pl.when(step == TC_GRID - 1)
    def _():
        o_ref[0, :] = jnp.sum(acc_ref[...], axis=1)


def _tc_sums(out, labels, importance):
    ofs = SC_B // BB
    return pl.pallas_call(
        _tc_kernel,
        grid=(TC_GRID,),
        in_specs=[
            pl.BlockSpec((BB, I, F), lambda g: (g + ofs, 0, 0)),
            pl.BlockSpec((BB, I, F), lambda g: (g + ofs, 0, 0)),
            pl.BlockSpec((I, F), lambda g: (0, 0)),
        ],
        out_specs=pl.BlockSpec((1, I), lambda g: (0, 0)),
        out_shape=jax.ShapeDtypeStruct((1, I), jnp.float32),
        scratch_shapes=[pltpu.VMEM((I, F), jnp.float32)],
    )(out, labels, importance)


def _fold_kernel(p_ref, t_ref, o_ref):
    s = jnp.sum(jnp.sum(p_ref[...], axis=0), axis=1) + t_ref[0, :]
    o_ref[0, :] = s * (1.0 / (B * F))


def _fold(partials, tcsums):
    return pl.pallas_call(
        _fold_kernel,
        in_specs=[
            pl.BlockSpec((NW, I, L), lambda: (0, 0, 0)),
            pl.BlockSpec((1, I), lambda: (0, 0)),
        ],
        out_specs=pl.BlockSpec((1, I), lambda: (0, 0)),
        out_shape=jax.ShapeDtypeStruct((1, I), jnp.float32),
    )(partials, tcsums)[0]


def kernel(out, labels, importance):
    o = out.reshape(B * I * F)
    l = labels.reshape(B * I * F)
    imp = importance.reshape(I * F)
    partials = _build_sc()(o, l, imp)
    tcsums = _tc_sums(out, labels, importance)
    return _fold(partials.reshape(NW, I, L), tcsums)


# speedup vs baseline: 3.3341x; 2.8241x over previous
import functools
import jax
import jax.numpy as jnp
from jax import lax
from jax.experimental import pallas as pl
from jax.experimental.pallas import tpu as pltpu
from jax.experimental.pallas import tpu_sc as plsc

B, I, F = 4096, 16, 512
L = 16
NC, NS = 2, 16
NW = NC * NS

SC_B = 768
TC_B = B - SC_B

BPW = SC_B // NW
CH_B = 2
NCHUNK = BPW // CH_B
NJ = F // L
assert BPW % CH_B == 0 and NCHUNK % 2 == 0

BB = 256
TC_GRID = TC_B // BB
assert TC_B % BB == 0 and SC_B % BB == 0


def _sc_body(o_hbm, l_hbm, imp_hbm, part_hbm,
             ob0, ob1, lb0, lb1, impv, accv,
             so0, so1, sl0, sl1):
    wid = lax.axis_index("s") * NC + lax.axis_index("c")
    base = wid * BPW

    pltpu.sync_copy(imp_hbm, impv)
    for k in range(I):
        accv[pl.ds(k * L, L)] = jnp.zeros((L,), jnp.float32)

    obufs, lbufs = (ob0, ob1), (lb0, lb1)
    osems, lsems = (so0, so1), (sl0, sl1)

    for s in range(2):
        boff = base + s * CH_B
        pltpu.async_copy(o_hbm.at[pl.ds(boff, CH_B)], obufs[s], osems[s])
        pltpu.async_copy(l_hbm.at[pl.ds(boff, CH_B)], lbufs[s], lsems[s])

    def pair_body(p, carry):
        for s in range(2):
            c = p * 2 + s
            boff = base + c * CH_B
            ob, lb = obufs[s], lbufs[s]
            pltpu.make_async_copy(o_hbm.at[pl.ds(boff, CH_B)], ob, osems[s]).wait()
            pltpu.make_async_copy(l_hbm.at[pl.ds(boff, CH_B)], lb, lsems[s]).wait()

            for b in range(CH_B):
                def i_body(i, _, ob=ob, lb=lb, b=b):
                    vacc = accv[pl.ds(i * L, L)]
                    for j in range(NJ):
                        o = ob[b, i, pl.ds(j * L, L)]
                        l = lb[b, i, pl.ds(j * L, L)]
                        m = impv[i, pl.ds(j * L, L)]
                        d = m * (jnp.abs(l) - o)
                        vacc = vacc + d * d
                    accv[pl.ds(i * L, L)] = vacc
                    return 0

                lax.fori_loop(0, I, i_body, 0)

            @pl.when(c + 2 < NCHUNK)
            def _():
                boff2 = base + (c + 2) * CH_B
                pltpu.async_copy(o_hbm.at[pl.ds(boff2, CH_B)], obufs[s], osems[s])
                pltpu.async_copy(l_hbm.at[pl.ds(boff2, CH_B)], lbufs[s], lsems[s])
        return carry

    lax.fori_loop(0, NCHUNK // 2, pair_body, 0)

    pltpu.sync_copy(accv, part_hbm.at[wid])


@functools.cache
def _build_sc():
    mesh = plsc.VectorSubcoreMesh(core_axis_name="c", subcore_axis_name="s")
    return pl.kernel(
        _sc_body,
        out_type=jax.ShapeDtypeStruct((NW, I * L), jnp.float32),
        mesh=mesh,
        scratch_types=[
            pltpu.VMEM((CH_B, I, F), jnp.float32),
            pltpu.VMEM((CH_B, I, F), jnp.float32),
            pltpu.VMEM((CH_B, I, F), jnp.float32),
            pltpu.VMEM((CH_B, I, F), jnp.float32),
            pltpu.VMEM((I, F), jnp.float32),
            pltpu.VMEM((I * L,), jnp.float32),
            pltpu.SemaphoreType.DMA,
            pltpu.SemaphoreType.DMA,
            pltpu.SemaphoreType.DMA,
            pltpu.SemaphoreType.DMA,
        ],
    )


def _tc_kernel(out_ref, lab_ref, imp_ref, o_ref, acc_ref):
    step = pl.program_id(0)
    d = imp_ref[...] * (jnp.abs(lab_ref[...]) - out_ref[...])
    partial = jnp.sum(d * d, axis=0)

    @pl.when(step == 0)
    def _():
        acc_ref[...] = partial

    @pl.when(step > 0)
    def _():
        acc_ref[...] = acc_ref[...] + partial

    @pl.when(step == TC_GRID - 1)
    def _():
        o_ref[0, :] = jnp.sum(acc_ref[...], axis=1)


def _tc_sums(out, labels, importance):
    ofs = SC_B // BB
    return pl.pallas_call(
        _tc_kernel,
        grid=(TC_GRID,),
        in_specs=[
            pl.BlockSpec((BB, I, F), lambda g: (g + ofs, 0, 0)),
            pl.BlockSpec((BB, I, F), lambda g: (g + ofs, 0, 0)),
            pl.BlockSpec((I, F), lambda g: (0, 0)),
        ],
        out_specs=pl.BlockSpec((1, I), lambda g: (0, 0)),
        out_shape=jax.ShapeDtypeStruct((1, I), jnp.float32),
        scratch_shapes=[pltpu.VMEM((I, F), jnp.float32)],
    )(out, labels, importance)


def _fold_kernel(p_ref, t_ref, o_ref):
    s = jnp.sum(jnp.sum(p_ref[...], axis=0), axis=1) + t_ref[0, :]
    o_ref[0, :] = s * (1.0 / (B * F))


def _fold(partials, tcsums):
    return pl.pallas_call(
        _fold_kernel,
        in_specs=[
            pl.BlockSpec((NW, I, L), lambda: (0, 0, 0)),
            pl.BlockSpec((1, I), lambda: (0, 0)),
        ],
        out_specs=pl.BlockSpec((1, I), lambda: (0, 0)),
        out_shape=jax.ShapeDtypeStruct((1, I), jnp.float32),
    )(partials, tcsums)[0]


def kernel(out, labels, importance):
    partials = _build_sc()(out, labels, importance)
    tcsums = _tc_sums(out, labels, importance)
    return _fold(partials.reshape(NW, I, L), tcsums)


# speedup vs baseline: 3.3785x; 1.0133x over previous
import functools
import jax
import jax.numpy as jnp
from jax import lax
from jax.experimental import pallas as pl
from jax.experimental.pallas import tpu as pltpu
from jax.experimental.pallas import tpu_sc as plsc

B, I, F = 4096, 16, 512
L = 16
NC, NS = 2, 16
NW = NC * NS

SC_B = 1024
TC_B = B - SC_B

BPW = SC_B // NW
CH_B = 2
NCHUNK = BPW // CH_B
NJ = F // L
assert BPW % CH_B == 0 and NCHUNK % 2 == 0

BB = 256
TC_GRID = TC_B // BB
assert TC_B % BB == 0 and SC_B % BB == 0


def _sc_body(o_hbm, l_hbm, imp_hbm, part_hbm,
             ob0, ob1, lb0, lb1, impv, accv,
             so0, so1, sl0, sl1):
    wid = lax.axis_index("s") * NC + lax.axis_index("c")
    base = wid * BPW

    pltpu.sync_copy(imp_hbm, impv)
    for k in range(I):
        accv[k, :] = jnp.zeros((L,), jnp.float32)

    obufs, lbufs = (ob0, ob1), (lb0, lb1)
    osems, lsems = (so0, so1), (sl0, sl1)

    for s in range(2):
        boff = base + s * CH_B
        pltpu.async_copy(o_hbm.at[pl.ds(boff, CH_B)], obufs[s], osems[s])
        pltpu.async_copy(l_hbm.at[pl.ds(boff, CH_B)], lbufs[s], lsems[s])

    def pair_body(p, carry):
        for s in range(2):
            c = p * 2 + s
            boff = base + c * CH_B
            ob, lb = obufs[s], lbufs[s]
            pltpu.make_async_copy(o_hbm.at[pl.ds(boff, CH_B)], ob, osems[s]).wait()
            pltpu.make_async_copy(l_hbm.at[pl.ds(boff, CH_B)], lb, lsems[s]).wait()

            def i_body(i, _, ob=ob, lb=lb):
                vacc = accv[i, :]
                for j in range(NJ):
                    m = impv[i, pl.ds(j * L, L)]
                    for b in range(CH_B):
                        o = ob[b, i, pl.ds(j * L, L)]
                        l = lb[b, i, pl.ds(j * L, L)]
                        d = m * (jnp.abs(l) - o)
                        vacc = vacc + d * d
                accv[i, :] = vacc
                return 0

            lax.fori_loop(0, I, i_body, 0)

            @pl.when(c + 2 < NCHUNK)
            def _():
                boff2 = base + (c + 2) * CH_B
                pltpu.async_copy(o_hbm.at[pl.ds(boff2, CH_B)], obufs[s], osems[s])
                pltpu.async_copy(l_hbm.at[pl.ds(boff2, CH_B)], lbufs[s], lsems[s])
        return carry

    lax.fori_loop(0, NCHUNK // 2, pair_body, 0)

    pltpu.sync_copy(accv, part_hbm.at[wid])


@functools.cache
def _build_sc():
    mesh = plsc.VectorSubcoreMesh(core_axis_name="c", subcore_axis_name="s")
    return pl.kernel(
        _sc_body,
        out_type=jax.ShapeDtypeStruct((NW, I, L), jnp.float32),
        mesh=mesh,
        scratch_types=[
            pltpu.VMEM((CH_B, I, F), jnp.float32),
            pltpu.VMEM((CH_B, I, F), jnp.float32),
            pltpu.VMEM((CH_B, I, F), jnp.float32),
            pltpu.VMEM((CH_B, I, F), jnp.float32),
            pltpu.VMEM((I, F), jnp.float32),
            pltpu.VMEM((I, L), jnp.float32),
            pltpu.SemaphoreType.DMA,
            pltpu.SemaphoreType.DMA,
            pltpu.SemaphoreType.DMA,
            pltpu.SemaphoreType.DMA,
        ],
    )


def _tc_kernel(out_ref, lab_ref, imp_ref, o_ref, acc_ref):
    step = pl.program_id(0)
    d = imp_ref[...] * (jnp.abs(lab_ref[...]) - out_ref[...])
    partial = jnp.sum(d * d, axis=0)

    @pl.when(step == 0)
    def _():
        acc_ref[...] = partial

    @pl.when(step > 0)
    def _():
        acc_ref[...] = acc_ref[...] + partial

    @pl.when(step == TC_GRID - 1)
    def _():
        o_ref[0, :] = jnp.sum(acc_ref[...], axis=1)


def _tc_sums(out, labels, importance):
    ofs = SC_B // BB
    return pl.pallas_call(
        _tc_kernel,
        grid=(TC_GRID,),
        in_specs=[
            pl.BlockSpec((BB, I, F), lambda g: (g + ofs, 0, 0)),
            pl.BlockSpec((BB, I, F), lambda g: (g + ofs, 0, 0)),
            pl.BlockSpec((I, F), lambda g: (0, 0)),
        ],
        out_specs=pl.BlockSpec((1, I), lambda g: (0, 0)),
        out_shape=jax.ShapeDtypeStruct((1, I), jnp.float32),
        scratch_shapes=[pltpu.VMEM((I, F), jnp.float32)],
    )(out, labels, importance)


def _fold_kernel(p_ref, t_ref, o_ref):
    s = jnp.sum(jnp.sum(p_ref[...], axis=0), axis=1) + t_ref[0, :]
    o_ref[0, :] = s * (1.0 / (B * F))


def _fold(partials, tcsums):
    return pl.pallas_call(
        _fold_kernel,
        in_specs=[
            pl.BlockSpec((NW, I, L), lambda: (0, 0, 0)),
            pl.BlockSpec((1, I), lambda: (0, 0)),
        ],
        out_specs=pl.BlockSpec((1, I), lambda: (0, 0)),
        out_shape=jax.ShapeDtypeStruct((1, I), jnp.float32),
    )(partials, tcsums)[0]


def kernel(out, labels, importance):
    partials = _build_sc()(out, labels, importance)
    tcsums = _tc_sums(out, labels, importance)
    return _fold(partials, tcsums)


# speedup vs baseline: 4.3200x; 1.2787x over previous
import jax
import jax.numpy as jnp
from jax import lax
from jax.experimental import pallas as pl
from jax.experimental.pallas import tpu as pltpu

B, I, F = 4096, 16, 512
CB = 64
NCHUNK = B // CB
NBUF = 4
NOUTER = NCHUNK // NBUF
assert NCHUNK % NBUF == 0


def _ring_kernel(o_hbm, l_hbm, imp_ref, o_ref, *scr):
    obufs = scr[0:NBUF]
    lbufs = scr[NBUF:2 * NBUF]
    osems = scr[2 * NBUF:3 * NBUF]
    lsems = scr[3 * NBUF:4 * NBUF]

    imp = imp_ref[...]

    for s in range(NBUF):
        boff = s * CB
        pltpu.async_copy(o_hbm.at[pl.ds(boff, CB)], obufs[s], osems[s])
        pltpu.async_copy(l_hbm.at[pl.ds(boff, CB)], lbufs[s], lsems[s])

    def outer_body(c0, acc):
        for s in range(NBUF):
            c = c0 * NBUF + s
            boff = c * CB
            ob, lb = obufs[s], lbufs[s]
            pltpu.make_async_copy(o_hbm.at[pl.ds(boff, CB)], ob, osems[s]).wait()
            pltpu.make_async_copy(l_hbm.at[pl.ds(boff, CB)], lb, lsems[s]).wait()

            def b_body(b, a, ob=ob, lb=lb):
                d = imp * (jnp.abs(lb[b]) - ob[b])
                return a + d * d

            acc = lax.fori_loop(0, CB, b_body, acc, unroll=2)

            @pl.when(c0 < NOUTER - 1)
            def _():
                boff2 = boff + NBUF * CB
                pltpu.async_copy(o_hbm.at[pl.ds(boff2, CB)], obufs[s], osems[s])
                pltpu.async_copy(l_hbm.at[pl.ds(boff2, CB)], lbufs[s], lsems[s])
        return acc

    acc = lax.fori_loop(0, NOUTER, outer_body,
                        jnp.zeros((I, F), jnp.float32))
    o_ref[0, :] = jnp.sum(acc, axis=1) * (1.0 / (B * F))


def kernel(out, labels, importance):
    scratch = (
        [pltpu.VMEM((CB, I, F), jnp.float32) for _ in range(2 * NBUF)]
        + [pltpu.SemaphoreType.DMA for _ in range(2 * NBUF)]
    )
    res = pl.pallas_call(
        _ring_kernel,
        in_specs=[
            pl.BlockSpec(memory_space=pl.ANY),
            pl.BlockSpec(memory_space=pl.ANY),
            pl.BlockSpec((I, F), lambda: (0, 0)),
        ],
        out_specs=pl.BlockSpec((1, I), lambda: (0, 0)),
        out_shape=jax.ShapeDtypeStruct((1, I), jnp.float32),
        scratch_shapes=scratch,
    )(out, labels, importance)
    return res[0]


# speedup vs baseline: 4.3262x; 1.0014x over previous
import jax
import jax.numpy as jnp
from jax import lax
from jax.experimental import pallas as pl
from jax.experimental.pallas import tpu as pltpu

B, I, F = 4096, 16, 512
CB = 32
NCHUNK = B // CB
NBUF = 8
NOUTER = NCHUNK // NBUF
assert NCHUNK % NBUF == 0


def _ring_kernel(o_hbm, l_hbm, imp_ref, o_ref, *scr):
    obufs = scr[0:NBUF]
    lbufs = scr[NBUF:2 * NBUF]
    osems = scr[2 * NBUF:3 * NBUF]
    lsems = scr[3 * NBUF:4 * NBUF]

    imp = imp_ref[...]

    for s in range(NBUF):
        boff = s * CB
        pltpu.async_copy(o_hbm.at[pl.ds(boff, CB)], obufs[s], osems[s])
        pltpu.async_copy(l_hbm.at[pl.ds(boff, CB)], lbufs[s], lsems[s])

    def outer_body(c0, acc):
        for s in range(NBUF):
            c = c0 * NBUF + s
            boff = c * CB
            ob, lb = obufs[s], lbufs[s]
            pltpu.make_async_copy(o_hbm.at[pl.ds(boff, CB)], ob, osems[s]).wait()
            pltpu.make_async_copy(l_hbm.at[pl.ds(boff, CB)], lb, lsems[s]).wait()

            def b_body(b, a, ob=ob, lb=lb):
                d = imp * (jnp.abs(lb[b]) - ob[b])
                return a + d * d

            acc = lax.fori_loop(0, CB, b_body, acc, unroll=2)

            @pl.when(c0 < NOUTER - 1)
            def _():
                boff2 = boff + NBUF * CB
                pltpu.async_copy(o_hbm.at[pl.ds(boff2, CB)], obufs[s], osems[s])
                pltpu.async_copy(l_hbm.at[pl.ds(boff2, CB)], lbufs[s], lsems[s])
        return acc

    acc = lax.fori_loop(0, NOUTER, outer_body,
                        jnp.zeros((I, F), jnp.float32))
    o_ref[0, :] = jnp.sum(acc, axis=1) * (1.0 / (B * F))


def kernel(out, labels, importance):
    scratch = (
        [pltpu.VMEM((CB, I, F), jnp.float32) for _ in range(2 * NBUF)]
        + [pltpu.SemaphoreType.DMA for _ in range(2 * NBUF)]
    )
    res = pl.pallas_call(
        _ring_kernel,
        in_specs=[
            pl.BlockSpec(memory_space=pl.ANY),
            pl.BlockSpec(memory_space=pl.ANY),
            pl.BlockSpec((I, F), lambda: (0, 0)),
        ],
        out_specs=pl.BlockSpec((1, I), lambda: (0, 0)),
        out_shape=jax.ShapeDtypeStruct((1, I), jnp.float32),
        scratch_shapes=scratch,
    )(out, labels, importance)
    return res[0]
